# baseline (speedup 1.0000x reference)
import jax
import jax.numpy as jnp
from jax import lax
from jax.experimental import pallas as pl
from jax.experimental.pallas import tpu as pltpu
from jax.experimental.pallas import tpu_sc as plsc

NPTS = 100
TAB = NPTS * NPTS
NC, NS, L = 2, 16, 16
NW = NC * NS
CHUNK = 2000


def _body(z_hbm, t_hbm, ctab_hbm, out_hbm,
          ctab_v, z_v, t_v, o_v,
          isem0, isem1, osem0, osem1, tabsem):
    n = z_hbm.shape[0]
    nchunks = n // CHUNK
    jmax = (nchunks + NW - 1) // NW
    isems = (isem0, isem1)
    osems = (osem0, osem1)
    wid = lax.axis_index("s") * NC + lax.axis_index("c")
    tab_copy = pltpu.make_async_copy(ctab_hbm, ctab_v, tabsem)
    tab_copy.start()

    def in_copies(j, b):
        k = wid + j * NW
        off = k * CHUNK
        return (
            pltpu.make_async_copy(z_hbm.at[pl.ds(off, CHUNK)],
                                  z_v.at[pl.ds(b * CHUNK, CHUNK)], isems[b]),
            pltpu.make_async_copy(t_hbm.at[pl.ds(off, CHUNK)],
                                  t_v.at[pl.ds(b * CHUNK, CHUNK)], isems[b]),
        )

    def out_copy(j, b):
        k = wid + j * NW
        off = k * CHUNK
        return pltpu.make_async_copy(o_v.at[pl.ds(b * CHUNK, CHUNK)],
                                     out_hbm.at[pl.ds(off, CHUNK)], osems[b])

    @pl.when(wid < nchunks)
    def _prime():
        for c in in_copies(0, 0):
            c.start()

    tab_copy.wait()

    @pl.loop(0, jmax, step=2)
    def _pair(j0):
        for b in range(2):
            j = j0 + b
            k = wid + j * NW

            @pl.when(k < nchunks)
            def _chunk():
                @pl.when(k + NW < nchunks)
                def _():
                    for c in in_copies(j + 1, 1 - b):
                        c.start()

                for c in in_copies(j, b):
                    c.wait()

                @pl.when(j >= 2)
                def _():
                    out_copy(j - 2, b).wait()

                boff = b * CHUNK

                @plsc.parallel_loop(0, CHUNK // L, unroll=2)
                def _vec(i):
                    s = boff + i * L
                    t = t_v[pl.ds(s, L)]
                    z = z_v[pl.ds(s, L)]
                    tc = t * 10.0 - 1.0
                    zc = z * 100.0 - 1.0
                    it = tc.astype(jnp.int32)
                    iz = zc.astype(jnp.int32)
                    ft = tc - it.astype(jnp.float32)
                    fz = zc - iz.astype(jnp.float32)
                    base = it * NPTS + iz
                    c00 = plsc.load_gather(ctab_v.at[pl.ds(0, TAB)], [base])
                    c01 = plsc.load_gather(ctab_v.at[pl.ds(TAB, TAB)], [base])
                    c10 = plsc.load_gather(ctab_v.at[pl.ds(2 * TAB, TAB)], [base])
                    c11 = plsc.load_gather(ctab_v.at[pl.ds(3 * TAB, TAB)], [base])
                    o_v[pl.ds(s, L)] = c00 + c01 * fz + (c10 + c11 * fz) * ft

                out_copy(j, b).start()

    jw = (nchunks - wid + NW - 1) // NW

    @pl.when(jw >= 1)
    def _():
        out_copy(0, 0).wait()

    @pl.when(jw >= 2)
    def _():
        out_copy(1, 1).wait()


def kernel(Z, T, flux_table):
    n = Z.shape[0]
    flat = jnp.pad(flux_table.reshape(-1), (0, NPTS + 1))
    v00 = flat[:TAB]
    v01 = flat[1:TAB + 1]
    v10 = flat[NPTS:TAB + NPTS]
    v11 = flat[NPTS + 1:TAB + NPTS + 1]
    c00 = v00
    c01 = v01 - v00
    c10 = v10 - v00
    c11 = (v11 - v10) - (v01 - v00)
    ctab = jnp.concatenate([c00, c01, c10, c11])

    mesh = plsc.VectorSubcoreMesh(core_axis_name="c", subcore_axis_name="s")
    f = pl.kernel(
        _body,
        out_type=jax.ShapeDtypeStruct((n,), jnp.float32),
        mesh=mesh,
        compiler_params=pltpu.CompilerParams(needs_layout_passes=False),
        scratch_types=[
            pltpu.VMEM((4 * TAB,), jnp.float32),
            pltpu.VMEM((2 * CHUNK,), jnp.float32),
            pltpu.VMEM((2 * CHUNK,), jnp.float32),
            pltpu.VMEM((2 * CHUNK,), jnp.float32),
            pltpu.SemaphoreType.DMA,
            pltpu.SemaphoreType.DMA,
            pltpu.SemaphoreType.DMA,
            pltpu.SemaphoreType.DMA,
            pltpu.SemaphoreType.DMA,
        ],
    )
    return f(Z, T, ctab)


# speedup vs baseline: 1.1412x; 1.1412x over previous
import jax
import jax.numpy as jnp
import numpy as np
from jax import lax
from jax.experimental import pallas as pl
from jax.experimental.pallas import tpu as pltpu
from jax.experimental.pallas import tpu_sc as plsc

NPTS = 100
TAB = NPTS * NPTS
NC, NS, L = 2, 16, 16
NW = NC * NS
CHUNK = 4000
TWO23 = np.float32(8388608.0)
MAGIC = np.float32(8388607.5)
KADD = np.int32(1761607680)


def _body(z_hbm, t_hbm, ctab_hbm, out_hbm,
          ctab_v, z_v, t_v, o_v,
          isem0, isem1, osem0, osem1, tabsem):
    n = z_hbm.shape[0]
    nchunks = n // CHUNK
    jmax = (nchunks + NW - 1) // NW
    isems = (isem0, isem1)
    osems = (osem0, osem1)
    wid = lax.axis_index("s") * NC + lax.axis_index("c")
    tab_copy = pltpu.make_async_copy(ctab_hbm, ctab_v, tabsem)
    tab_copy.start()

    def in_copies(j, b):
        k = wid + j * NW
        off = k * CHUNK
        return (
            pltpu.make_async_copy(z_hbm.at[pl.ds(off, CHUNK)],
                                  z_v.at[pl.ds(b * CHUNK, CHUNK)], isems[b]),
            pltpu.make_async_copy(t_hbm.at[pl.ds(off, CHUNK)],
                                  t_v.at[pl.ds(b * CHUNK, CHUNK)], isems[b]),
        )

    def out_copy(j, b):
        k = wid + j * NW
        off = k * CHUNK
        return pltpu.make_async_copy(o_v.at[pl.ds(b * CHUNK, CHUNK)],
                                     out_hbm.at[pl.ds(off, CHUNK)], osems[b])

    @pl.when(wid < nchunks)
    def _prime():
        for c in in_copies(0, 0):
            c.start()

    tab_copy.wait()

    @pl.loop(0, jmax, step=2)
    def _pair(j0):
        for b in range(2):
            j = j0 + b
            k = wid + j * NW

            @pl.when(k < nchunks)
            def _chunk():
                @pl.when(k + NW < nchunks)
                def _():
                    for c in in_copies(j + 1, 1 - b):
                        c.start()

                for c in in_copies(j, b):
                    c.wait()

                @pl.when(j >= 2)
                def _():
                    out_copy(j - 2, b).wait()

                boff = b * CHUNK

                @plsc.parallel_loop(0, CHUNK // L, unroll=2)
                def _vec(i):
                    s = boff + i * L
                    t = t_v[pl.ds(s, L)]
                    z = z_v[pl.ds(s, L)]
                    ut = t * 10.0
                    uz = z * 100.0
                    yt = ut + MAGIC
                    yz = uz + MAGIC
                    ft = ut - (yt - TWO23)
                    fz = uz - (yz - TWO23)
                    base = (plsc.bitcast(yt, jnp.int32) * NPTS
                            + plsc.bitcast(yz, jnp.int32) + KADD)
                    c00 = plsc.load_gather(ctab_v.at[pl.ds(0, TAB)], [base])
                    c01 = plsc.load_gather(ctab_v.at[pl.ds(TAB, TAB)], [base])
                    c10 = plsc.load_gather(ctab_v.at[pl.ds(2 * TAB, TAB)], [base])
                    c11 = plsc.load_gather(ctab_v.at[pl.ds(3 * TAB, TAB)], [base])
                    o_v[pl.ds(s, L)] = c00 + c01 * fz + (c10 + c11 * fz) * ft

                out_copy(j, b).start()

    jw = (nchunks - wid + NW - 1) // NW

    @pl.when(jw >= 1)
    def _():
        out_copy(0, 0).wait()

    @pl.when(jw >= 2)
    def _():
        out_copy(1, 1).wait()


def kernel(Z, T, flux_table):
    n = Z.shape[0]
    ext = jnp.pad(flux_table, ((1, 0), (1, 0)), mode="edge")
    a = ext[:NPTS, :NPTS].reshape(-1)
    bq = ext[:NPTS, 1:].reshape(-1)
    cq = ext[1:, :NPTS].reshape(-1)
    dq = ext[1:, 1:].reshape(-1)
    c00 = a
    c01 = bq - a
    c10 = cq - a
    c11 = (dq - cq) - (bq - a)
    ctab = jnp.concatenate([c00, c01, c10, c11])

    mesh = plsc.VectorSubcoreMesh(core_axis_name="c", subcore_axis_name="s")
    f = pl.kernel(
        _body,
        out_type=jax.ShapeDtypeStruct((n,), jnp.float32),
        mesh=mesh,
        compiler_params=pltpu.CompilerParams(needs_layout_passes=False),
        scratch_types=[
            pltpu.VMEM((4 * TAB,), jnp.float32),
            pltpu.VMEM((2 * CHUNK,), jnp.float32),
            pltpu.VMEM((2 * CHUNK,), jnp.float32),
            pltpu.VMEM((2 * CHUNK,), jnp.float32),
            pltpu.SemaphoreType.DMA,
            pltpu.SemaphoreType.DMA,
            pltpu.SemaphoreType.DMA,
            pltpu.SemaphoreType.DMA,
            pltpu.SemaphoreType.DMA,
        ],
    )
    return f(Z, T, ctab)
